# baseline (speedup 1.0000x reference)
import functools

import jax
import jax.numpy as jnp
from jax import lax
from jax.experimental import pallas as pl
from jax.experimental.pallas import tpu as pltpu
from jax.experimental.pallas import tpu_sc as plsc

_NC = 2
_NS = 16
_NW = _NC * _NS
_CS = 8
_RING = 3
_LANES = 16
_VUNROLL = 64


def _make_sc_kernel(S, B, D):
    s_per_w = S // _NW
    n_chunks = s_per_w // _CS
    n_vec = D // _LANES
    mesh = plsc.VectorSubcoreMesh(
        core_axis_name="c", subcore_axis_name="s",
        num_cores=_NC, num_subcores=_NS)

    @functools.partial(
        pl.kernel,
        mesh=mesh,
        out_type=jax.ShapeDtypeStruct((S, B, D), jnp.float32),
        scratch_types=(
            [pltpu.VMEM((_CS, B, D), jnp.float32) for _ in range(_RING)]
            + [pltpu.VMEM((_CS, D), jnp.float32) for _ in range(_RING)]
            + [pltpu.SemaphoreType.DMA for _ in range(3 * _RING)]
        ),
    )
    def k(x_hbm, table_hbm, out_hbm, *scratch):
        xbufs = scratch[:_RING]
        ebufs = scratch[_RING:2 * _RING]
        sems = scratch[2 * _RING:]
        inx_sems = sems[:_RING]
        ine_sems = sems[_RING:2 * _RING]
        out_sems = sems[2 * _RING:]

        wid = lax.axis_index("s") * _NC + lax.axis_index("c")
        s_base = wid * s_per_w

        def start_in(c):
            s0 = s_base + c * _CS
            r = c % _RING
            dx = pltpu.async_copy(
                x_hbm.at[pl.ds(s0, _CS)], xbufs[r], inx_sems[r])
            de = pltpu.async_copy(
                table_hbm.at[pl.ds(s0, _CS)], ebufs[r], ine_sems[r])
            return dx, de

        def start_out(c):
            s0 = s_base + c * _CS
            r = c % _RING
            return pltpu.async_copy(
                xbufs[r], out_hbm.at[pl.ds(s0, _CS)], out_sems[r])

        in_d = {c: start_in(c) for c in range(2)}
        out_d = {}
        for c in range(n_chunks):
            r = c % _RING
            dx, de = in_d.pop(c)
            dx.wait()
            de.wait()
            xbuf, ebuf = xbufs[r], ebufs[r]

            @pl.loop(0, _CS)
            def _row(i):
                @pl.loop(0, n_vec, unroll=_VUNROLL)
                def _vec(v):
                    ev = ebuf[i, pl.ds(v * _LANES, _LANES)]
                    for b in range(B):
                        plsc.addupdate(
                            xbuf.at[i, b, pl.ds(v * _LANES, _LANES)], ev)

            out_d[c] = start_out(c)
            nxt = c + 2
            if nxt < n_chunks:
                prev = nxt - _RING
                if prev >= 0:
                    out_d.pop(prev).wait()
                in_d[nxt] = start_in(nxt)
        for c in sorted(out_d):
            out_d[c].wait()

    return k


def kernel(x, pos_emb_table):
    S, B, D = x.shape
    return _make_sc_kernel(S, B, D)(x, pos_emb_table[:S])


# speedup vs baseline: 1.1402x; 1.1402x over previous
import functools

import jax
import jax.numpy as jnp
from jax import lax
from jax.experimental import pallas as pl
from jax.experimental.pallas import tpu as pltpu
from jax.experimental.pallas import tpu_sc as plsc

_NC = 2
_NS = 16
_NW = _NC * _NS
_CS = 8
_RING = 3
_LANES = 16
_VGROUP = 8


def _make_sc_kernel(S, B, D):
    s_per_w = S // _NW
    n_chunks = s_per_w // _CS
    n_vec = D // _LANES
    mesh = plsc.VectorSubcoreMesh(
        core_axis_name="c", subcore_axis_name="s",
        num_cores=_NC, num_subcores=_NS)

    @functools.partial(
        pl.kernel,
        mesh=mesh,
        out_type=jax.ShapeDtypeStruct((S, B, D), jnp.float32),
        scratch_types=(
            [pltpu.VMEM((_CS, B, D), jnp.float32) for _ in range(_RING)]
            + [pltpu.VMEM((_CS, D), jnp.float32) for _ in range(_RING)]
            + [pltpu.SemaphoreType.DMA for _ in range(3 * _RING)]
        ),
    )
    def k(x_hbm, table_hbm, out_hbm, *scratch):
        xbufs = scratch[:_RING]
        ebufs = scratch[_RING:2 * _RING]
        sems = scratch[2 * _RING:]
        inx_sems = sems[:_RING]
        ine_sems = sems[_RING:2 * _RING]
        out_sems = sems[2 * _RING:]

        wid = lax.axis_index("s") * _NC + lax.axis_index("c")
        s_base = wid * s_per_w

        def start_in(c):
            s0 = s_base + c * _CS
            r = c % _RING
            dx = pltpu.async_copy(
                x_hbm.at[pl.ds(s0, _CS)], xbufs[r], inx_sems[r])
            de = pltpu.async_copy(
                table_hbm.at[pl.ds(s0, _CS)], ebufs[r], ine_sems[r])
            return dx, de

        def start_out(c):
            s0 = s_base + c * _CS
            r = c % _RING
            return pltpu.async_copy(
                xbufs[r], out_hbm.at[pl.ds(s0, _CS)], out_sems[r])

        in_d = {c: start_in(c) for c in range(2)}
        out_d = {}
        for c in range(n_chunks):
            r = c % _RING
            dx, de = in_d.pop(c)
            dx.wait()
            de.wait()
            xbuf, ebuf = xbufs[r], ebufs[r]

            @pl.loop(0, _CS)
            def _row(i):
                @pl.loop(0, n_vec // _VGROUP)
                def _vec(g):
                    evs = [ebuf[i, pl.ds((g * _VGROUP + j) * _LANES, _LANES)]
                           for j in range(_VGROUP)]
                    for j in range(_VGROUP):
                        for b in range(B):
                            plsc.addupdate(
                                xbuf.at[i, b,
                                        pl.ds((g * _VGROUP + j) * _LANES,
                                              _LANES)],
                                evs[j])

            out_d[c] = start_out(c)
            nxt = c + 2
            if nxt < n_chunks:
                prev = nxt - _RING
                if prev >= 0:
                    out_d.pop(prev).wait()
                in_d[nxt] = start_in(nxt)
        for c in sorted(out_d):
            out_d[c].wait()

    return k


def kernel(x, pos_emb_table):
    S, B, D = x.shape
    return _make_sc_kernel(S, B, D)(x, pos_emb_table[:S])
